# baseline (speedup 1.0000x reference)
import functools
import jax
import jax.numpy as jnp
from jax import lax
from jax.experimental import pallas as pl
from jax.experimental.pallas import tpu as pltpu
from jax.experimental.pallas import tpu_sc as plsc

N = 10000
E = 320000
NC = 2
NS = 16
K = 80

_MESH = dict(
    mesh=plsc.VectorSubcoreMesh(
        core_axis_name="c", subcore_axis_name="s", num_cores=NC, num_subcores=NS
    )
)


@functools.partial(
    pl.kernel,
    out_type=jax.ShapeDtypeStruct((2, 1, N), jnp.float32),
    scratch_types=[
        pltpu.VMEM((250, K), jnp.int32),
        pltpu.VMEM((K,), jnp.float32),
        pltpu.VMEM((2000,), jnp.float32),
        pltpu.VMEM_SHARED((N,), jnp.float32),
        pltpu.SemaphoreType.DMA,
    ],
    **_MESH,
)
def _sc_degrees(eidx, out, idx_all, ones_v, zbuf, acc, sem):
    cid = lax.axis_index("c")
    sid = lax.axis_index("s")

    for j in range(K // 16):
        ones_v[pl.ds(j * 16, 16)] = jnp.ones((16,), jnp.float32)

    @pl.when(sid == 0)
    def _zero():
        def zrow(i, _):
            zbuf[pl.ds(i * 16, 16)] = jnp.zeros((16,), jnp.float32)
            return 0
        lax.fori_loop(0, 2000 // 16, zrow, 0)
        for j in range(N // 2000):
            pltpu.sync_copy(zbuf, acc.at[pl.ds(j * 2000, 2000)])

    pltpu.sync_copy(eidx.at[cid, sid], idx_all)

    plsc.subcore_barrier()

    def outer(o, _):
        ds = []
        for b in range(10):
            g = o * 10 + b
            ds.append(pltpu.async_copy(ones_v, acc.at[idx_all.at[g]], sem, add=True))
        for d in ds:
            d.wait()
        return 0

    lax.fori_loop(0, 25, outer, 0)

    plsc.subcore_barrier()

    @pl.when(sid == 0)
    def _writeout():
        pltpu.sync_copy(acc, out.at[cid, 0])


_PER_W = E // (NC * NS)
KA = 128
NCH = (_PER_W + KA - 1) // KA
EPW = NCH * KA
NA = N + 8
_NI = 6
_NR = 2


@functools.partial(
    pl.kernel,
    out_type=jax.ShapeDtypeStruct((NC, N, 128), jnp.float32),
    scratch_types=[
        pltpu.VMEM((_NI, KA), jnp.int32),
        pltpu.VMEM((_NI, KA), jnp.int32),
        pltpu.VMEM((_NR, KA, 128), jnp.float32),
        pltpu.VMEM_SHARED((NA, 128), jnp.float32),
        pltpu.SemaphoreType.DMA((_NI,)),
        pltpu.SemaphoreType.DMA((_NR,)),
        pltpu.SemaphoreType.DMA((_NR,)),
    ],
    **_MESH,
)
def _sc_agg(table, sidx, didx, out, idx_sr, idx_dr, rows, acc, sem_i, sem_g,
            sem_s):
    cid = lax.axis_index("c")
    sid = lax.axis_index("s")
    w = cid * NS + sid

    def zrow(i, _):
        for j in range(8):
            rows[0, i, pl.ds(j * 16, 16)] = jnp.zeros((16,), jnp.float32)
        return 0
    lax.fori_loop(0, 40, zrow, 0)

    @pl.when(sid < 10)
    def _zero():
        def zblk(j, _):
            pltpu.sync_copy(rows.at[0, pl.ds(0, 40)],
                            acc.at[pl.ds(sid * 1000 + j * 40, 40)])
            return 0
        lax.fori_loop(0, 25, zblk, 0)

    def idx_start(g, b):
        pltpu.async_copy(sidx.at[w, pl.ds(g, 1)], idx_sr.at[pl.ds(b, 1)],
                         sem_i.at[b])
        pltpu.async_copy(didx.at[w, pl.ds(g, 1)], idx_dr.at[pl.ds(b, 1)],
                         sem_i.at[b])

    def idx_wait(g, b):
        pltpu.make_async_copy(sidx.at[w, pl.ds(g, 1)], idx_sr.at[pl.ds(b, 1)],
                              sem_i.at[b]).wait()
        pltpu.make_async_copy(didx.at[w, pl.ds(g, 1)], idx_dr.at[pl.ds(b, 1)],
                              sem_i.at[b]).wait()

    def gather_start(b, r):
        pltpu.async_copy(table.at[idx_sr.at[b]], rows.at[r], sem_g.at[r])

    def gather_wait(b, r):
        pltpu.make_async_copy(table.at[idx_sr.at[b]], rows.at[r],
                              sem_g.at[r]).wait()

    def scatter_start(b, r):
        pltpu.async_copy(rows.at[r], acc.at[idx_dr.at[b]], sem_s.at[r],
                         add=True)

    def scatter_wait(b, r):
        pltpu.make_async_copy(rows.at[r], acc.at[idx_dr.at[b]],
                              sem_s.at[r]).wait()

    for g0 in range(4):
        idx_start(g0, g0)

    plsc.subcore_barrier()

    def body(o, _):
        for b in range(_NI):
            g = o * _NI + b
            r = b % _NR

            @pl.when(g >= 2)
            def _ret_sc():
                scatter_wait((b - 2) % _NI, (b - 2) % _NR)

            @pl.when(g + 4 < NCH)
            def _pref():
                idx_start(g + 4, (b + 4) % _NI)

            idx_wait(g, b)
            gather_start(b, r)

            @pl.when(g >= 1)
            def _sc_prev():
                gather_wait((b - 1) % _NI, (b - 1) % _NR)
                scatter_start((b - 1) % _NI, (b - 1) % _NR)
        return 0

    lax.fori_loop(0, (NCH - 1) // _NI, body, 0)

    gl = NCH - 1
    bl = gl % _NI
    rl = gl % _NR
    scatter_wait((bl - 2) % _NI, (gl - 2) % _NR)
    idx_wait(gl, bl)
    gather_start(bl, rl)
    gather_wait((bl - 1) % _NI, (gl - 1) % _NR)
    scatter_start((bl - 1) % _NI, (gl - 1) % _NR)
    gather_wait(bl, rl)
    scatter_start(bl, rl)
    scatter_wait((bl - 1) % _NI, (gl - 1) % _NR)
    scatter_wait(bl, rl)

    plsc.subcore_barrier()

    @pl.when(sid < 10)
    def _writeout():
        pltpu.sync_copy(
            acc.at[pl.ds(sid * 1000, 1000)],
            out.at[cid, pl.ds(sid * 1000, 1000)],
        )


_BR = 400


def _prep_body(x_ref, ds_ref, dd_ref, h0_ref, ns_ref, nd_ref):
    ns = lax.rsqrt(jnp.maximum(ds_ref[...], 1.0))
    nd = lax.rsqrt(jnp.maximum(dd_ref[...], 1.0))
    h0_ref[...] = x_ref[...] * ns
    ns_ref[...] = ns
    nd_ref[...] = nd


def _prep(x, ds_col, dd_col):
    return pl.pallas_call(
        _prep_body,
        grid=(N // _BR,),
        in_specs=[
            pl.BlockSpec((_BR, 128), lambda i: (i, 0)),
            pl.BlockSpec((_BR, 1), lambda i: (i, 0)),
            pl.BlockSpec((_BR, 1), lambda i: (i, 0)),
        ],
        out_specs=[
            pl.BlockSpec((_BR, 128), lambda i: (i, 0)),
            pl.BlockSpec((_BR, 1), lambda i: (i, 0)),
            pl.BlockSpec((_BR, 1), lambda i: (i, 0)),
        ],
        out_shape=[
            jax.ShapeDtypeStruct((N, 128), jnp.float32),
            jax.ShapeDtypeStruct((N, 1), jnp.float32),
            jax.ShapeDtypeStruct((N, 1), jnp.float32),
        ],
    )(x, ds_col, dd_col)


def _mid_body(p0_ref, p1_ref, nd_ref, ns_ref, w1_ref, b1_ref, w2_ref, o_ref):
    agg = (p0_ref[...] + p1_ref[...]) * nd_ref[...]
    h = jnp.dot(agg, w1_ref[...], preferred_element_type=jnp.float32)
    h = jnp.maximum(h + b1_ref[...], 0.0)
    h2 = jnp.dot(h, w2_ref[...], preferred_element_type=jnp.float32)
    o_ref[...] = h2 * ns_ref[...]


def _mid(p0, p1, nd, ns, W1, b1r, W2):
    return pl.pallas_call(
        _mid_body,
        grid=(N // _BR,),
        in_specs=[
            pl.BlockSpec((_BR, 128), lambda i: (i, 0)),
            pl.BlockSpec((_BR, 128), lambda i: (i, 0)),
            pl.BlockSpec((_BR, 1), lambda i: (i, 0)),
            pl.BlockSpec((_BR, 1), lambda i: (i, 0)),
            pl.BlockSpec((128, 256), lambda i: (0, 0)),
            pl.BlockSpec((1, 256), lambda i: (0, 0)),
            pl.BlockSpec((256, 128), lambda i: (0, 0)),
        ],
        out_specs=pl.BlockSpec((_BR, 128), lambda i: (i, 0)),
        out_shape=jax.ShapeDtypeStruct((N, 128), jnp.float32),
    )(p0, p1, nd, ns, W1, b1r, W2)


def _post_body(p0_ref, p1_ref, nd_ref, b2_ref, o_ref):
    agg = (p0_ref[...] + p1_ref[...]) * nd_ref[...]
    o_ref[...] = jnp.maximum(agg + b2_ref[...], 0.0)


def _post(p0, p1, nd, b2r):
    return pl.pallas_call(
        _post_body,
        grid=(N // _BR,),
        in_specs=[
            pl.BlockSpec((_BR, 128), lambda i: (i, 0)),
            pl.BlockSpec((_BR, 128), lambda i: (i, 0)),
            pl.BlockSpec((_BR, 1), lambda i: (i, 0)),
            pl.BlockSpec((1, 128), lambda i: (0, 0)),
        ],
        out_specs=pl.BlockSpec((_BR, 128), lambda i: (i, 0)),
        out_shape=jax.ShapeDtypeStruct((N, 128), jnp.float32),
    )(p0, p1, nd, b2r)


def kernel(x, edge_index, W1, b1, W2, b2):
    eidx = edge_index.astype(jnp.int32)
    npad = EPW - _PER_W
    src_p = jnp.concatenate(
        [eidx[0].reshape(NC * NS, _PER_W),
         jnp.zeros((NC * NS, npad), jnp.int32)], axis=1
    ).reshape(NC * NS, NCH, KA)
    dst_p = jnp.concatenate(
        [eidx[1].reshape(NC * NS, _PER_W),
         jnp.full((NC * NS, npad), N, jnp.int32)], axis=1
    ).reshape(NC * NS, NCH, KA)

    deg = _sc_degrees(eidx.reshape(2, NS, E // NS // K, K))
    ds_col = deg[0, 0].reshape(N, 1)
    dd_col = deg[1, 0].reshape(N, 1)

    h0, ns, nd = _prep(x, ds_col, dd_col)

    p = _sc_agg(h0, src_p, dst_p)
    h1s = _mid(p[0], p[1], nd, ns, W1, b1.reshape(1, -1), W2)

    q = _sc_agg(h1s, src_p, dst_p)
    return _post(q[0], q[1], nd, b2.reshape(1, -1))
